# baseline (speedup 1.0000x reference)
import functools

import jax
import jax.numpy as jnp
from jax import lax
from jax.experimental import pallas as pl
from jax.experimental.pallas import tpu as pltpu
from jax.experimental.pallas import tpu_sc as plsc

NC = 2
NS = 16
NW = NC * NS
L = 16
NB = 64


@functools.lru_cache(maxsize=None)
def _make_kernel(B: int, F: int, D: int, NUM: int):
    OW = F * D + NUM
    CH = NB * F
    Btot = B * F
    per_w = Btot // NW
    per_wb = B // NW
    nchunk = per_wb // NB
    ngrp = CH // L
    assert per_w * NW == Btot and nchunk * NB == per_wb and ngrp * L == CH
    assert D == L

    mesh = plsc.VectorSubcoreMesh(core_axis_name="c", subcore_axis_name="s")

    @functools.partial(
        pl.kernel,
        out_type=jax.ShapeDtypeStruct((B * OW,), jnp.float32),
        mesh=mesh,
        compiler_params=pltpu.CompilerParams(use_tc_tiling_on_sc=False),
        scratch_types=[
            pltpu.VMEM((CH,), jnp.int32),
            pltpu.VMEM((CH,), jnp.int32),
            pltpu.VMEM((CH,), jnp.int32),
            pltpu.VMEM((CH, L), jnp.float32),
            pltpu.VMEM((NB * NUM + L,), jnp.float32),
            pltpu.VMEM((NB * OW + L,), jnp.float32),
            pltpu.SemaphoreType.DMA,
        ],
    )
    def k(cat_hbm, offs_hbm, numf_hbm, tab_hbm, out_hbm,
          idx_v, offs_v, flat_v, rows_v, num_v, outrow_v, sem):
        wid = lax.axis_index("s") * NC + lax.axis_index("c")
        tile_rbase = wid * per_w
        tile_bbase = wid * per_wb
        pltpu.sync_copy(offs_hbm, offs_v)
        zero16 = jnp.zeros((L,), jnp.float32)

        def chunk(c, carry):
            rbase = tile_rbase + c * CH
            b0 = tile_bbase + c * NB
            pltpu.sync_copy(cat_hbm.at[pl.ds(rbase, CH)], idx_v)
            pltpu.sync_copy(
                numf_hbm.at[pl.ds(b0 * NUM, NB * NUM)],
                num_v.at[pl.ds(0, NB * NUM)],
            )
            for g in range(ngrp):
                flat_v[pl.ds(g * L, L)] = (
                    idx_v[pl.ds(g * L, L)] + offs_v[pl.ds(g * L, L)])
            gcp = pltpu.async_copy(tab_hbm.at[flat_v], rows_v, sem)

            def nump(b, c2):
                n16 = num_v[pl.ds(b * NUM, L)]
                outrow_v[pl.ds(b * OW + F * D, L)] = n16
                return c2

            lax.fori_loop(0, NB, nump, 0)
            gcp.wait()

            def rp(g, c2):
                i16 = idx_v[pl.ds(g * L, L)]
                for j in range(L):
                    r = g * L + j
                    b = r // F
                    f = r - b * F
                    val = jnp.where(i16[j] == 0, zero16, rows_v[r])
                    outrow_v[pl.ds(b * OW + f * D, L)] = val
                return c2

            lax.fori_loop(0, ngrp, rp, 0)
            pltpu.sync_copy(
                outrow_v.at[pl.ds(0, NB * OW)],
                out_hbm.at[pl.ds(b0 * OW, NB * OW)],
            )
            return carry

        lax.fori_loop(0, nchunk, chunk, 0)

    return k


def kernel(numerical, categorical, tables):
    B, NUM = numerical.shape
    _, F = categorical.shape
    _, V, D = tables.shape
    CH = NB * F

    cat_flat = categorical.reshape(B * F)
    offs = (jnp.arange(CH, dtype=jnp.int32) % F) * V
    numf = numerical.reshape(B * NUM)
    one = numerical[0, 0] * 0.0 + 1.0
    tab_flat = (tables * one).reshape(F * V, D)

    out = _make_kernel(B, F, D, NUM)(cat_flat, offs, numf, tab_flat)
    return out.reshape(B, F * D + NUM)


# speedup vs baseline: 3.0193x; 3.0193x over previous
import functools

import jax
import jax.numpy as jnp
from jax import lax
from jax.experimental import pallas as pl
from jax.experimental.pallas import tpu as pltpu
from jax.experimental.pallas import tpu_sc as plsc

NC = 2
NS = 16
NW = NC * NS
L = 16
NB = 64


@functools.lru_cache(maxsize=None)
def _make_kernel(B: int, F: int, V: int, D: int, NUM: int):
    OW = F * D + NUM
    CH = NB * F
    Btot = B * F
    per_w = Btot // NW
    per_wb = B // NW
    nchunk = per_wb // NB
    ngrp = CH // L
    assert per_w * NW == Btot and nchunk * NB == per_wb and ngrp * L == CH
    assert D == L

    mesh = plsc.VectorSubcoreMesh(core_axis_name="c", subcore_axis_name="s")

    @functools.partial(
        pl.kernel,
        out_type=jax.ShapeDtypeStruct((B * OW,), jnp.float32),
        mesh=mesh,
        compiler_params=pltpu.CompilerParams(use_tc_tiling_on_sc=False),
        scratch_types=[
            pltpu.VMEM((CH,), jnp.int32),
            pltpu.VMEM((CH,), jnp.int32),
            pltpu.VMEM((CH * L,), jnp.int32),
            pltpu.VMEM((CH * L,), jnp.float32),
            pltpu.VMEM((NB * NUM + L,), jnp.float32),
            pltpu.VMEM((NB * OW + L,), jnp.float32),
            pltpu.SemaphoreType.DMA,
        ],
    )
    def k(cat_hbm, foffs_hbm, numf_hbm, tabw_hbm, out_hbm,
          idx_v, foffs_v, widx_v, gath_v, num_v, outrow_v, sem):
        wid = lax.axis_index("s") * NC + lax.axis_index("c")
        tile_rbase = wid * per_w
        tile_bbase = wid * per_wb
        pltpu.sync_copy(foffs_hbm, foffs_v)
        zero16 = jnp.zeros((L,), jnp.float32)
        dtimesv = lax.iota(jnp.int32, L) * jnp.int32(V)

        def chunk(c, carry):
            rbase = tile_rbase + c * CH
            b0 = tile_bbase + c * NB
            pltpu.sync_copy(cat_hbm.at[pl.ds(rbase, CH)], idx_v)
            pltpu.sync_copy(
                numf_hbm.at[pl.ds(b0 * NUM, NB * NUM)],
                num_v.at[pl.ds(0, NB * NUM)],
            )

            def mkwidx(g, c2):
                base16 = idx_v[pl.ds(g * L, L)] + foffs_v[pl.ds(g * L, L)]
                for j in range(L):
                    r = g * L + j
                    widx_v[pl.ds(r * L, L)] = base16[j] + dtimesv
                return c2

            lax.fori_loop(0, ngrp, mkwidx, 0)
            gcp = pltpu.async_copy(tabw_hbm.at[widx_v], gath_v, sem)

            def nump(b, c2):
                n16 = num_v[pl.ds(b * NUM, L)]
                outrow_v[pl.ds(b * OW + F * D, L)] = n16
                return c2

            lax.fori_loop(0, NB, nump, 0)
            gcp.wait()

            def rp(g, c2):
                i16 = idx_v[pl.ds(g * L, L)]
                for j in range(L):
                    r = g * L + j
                    b = r // F
                    f = r - b * F
                    val = jnp.where(
                        i16[j] == 0, zero16, gath_v[pl.ds(r * L, L)])
                    outrow_v[pl.ds(b * OW + f * D, L)] = val
                return c2

            lax.fori_loop(0, ngrp, rp, 0)
            pltpu.sync_copy(
                outrow_v.at[pl.ds(0, NB * OW)],
                out_hbm.at[pl.ds(b0 * OW, NB * OW)],
            )
            return carry

        lax.fori_loop(0, nchunk, chunk, 0)

    return k


def kernel(numerical, categorical, tables):
    B, NUM = numerical.shape
    _, F = categorical.shape
    _, V, D = tables.shape
    CH = NB * F

    cat_flat = categorical.reshape(B * F)
    foffs = (jnp.arange(CH, dtype=jnp.int32) % F) * (D * V)
    numf = numerical.reshape(B * NUM)
    tabw = jnp.swapaxes(tables, 1, 2).reshape(F * D * V)

    out = _make_kernel(B, F, V, D, NUM)(cat_flat, foffs, numf, tabw)
    return out.reshape(B, F * D + NUM)
